# Initial kernel scaffold; baseline (speedup 1.0000x reference)
import jax
import jax.numpy as jnp
from jax.experimental import pallas as pl


def kernel(hidden_states, gate_w, Wg, Wu, Wd, Sg, Su, Sd):
    raise NotImplementedError("write your pallas kernel here")


# speedup vs baseline: 1.1525x; 1.1525x over previous
import functools

import jax
import jax.numpy as jnp
from jax.experimental import pallas as pl
from jax.experimental.pallas import tpu as pltpu

_ALPHA = 0.01
_NEG_INF = -1e30


def _router_body(x_ref, gwt_ref, cmb_ref, aux_ref, *, n_experts, k_top):
    x = x_ref[...]
    gwt = gwt_ref[...]
    n = x.shape[0]
    logits = jax.lax.dot_general(
        x, gwt, (((1,), (0,)), ((), ())),
        preferred_element_type=jnp.float32,
    )
    cols = jax.lax.broadcasted_iota(jnp.int32, logits.shape, 1)
    valid = cols < n_experts
    masked = jnp.where(valid, logits, _NEG_INF)
    m = jnp.max(masked, axis=-1, keepdims=True)
    p = jnp.where(valid, jnp.exp(masked - m), 0.0)
    z = jnp.sum(p, axis=-1, keepdims=True)
    scores = p / z
    m1 = jnp.max(masked, axis=-1, keepdims=True)
    i1 = jnp.min(jnp.where(masked == m1, cols, 127), axis=-1, keepdims=True)
    masked2 = jnp.where(cols == i1, _NEG_INF, masked)
    m2 = jnp.max(masked2, axis=-1, keepdims=True)
    i2 = jnp.min(jnp.where(masked2 == m2, cols, 127), axis=-1, keepdims=True)
    e1 = jnp.exp(m1 - m)
    e2 = jnp.exp(m2 - m)
    w1 = e1 / (e1 + e2)
    w2 = e2 / (e1 + e2)
    cmb = jnp.where(cols == i1, w1, 0.0) + jnp.where(cols == i2, w2, 0.0)
    cmb_ref[...] = cmb
    picks = (cols == i1).astype(jnp.float32) + (cols == i2).astype(jnp.float32)
    counts = jnp.sum(picks, axis=0)
    sum_scores = jnp.sum(scores, axis=0)
    aux = _ALPHA * n_experts * jnp.sum(
        (counts / (n * k_top)) * (sum_scores / n))
    aux_ref[0, 0] = aux


def _expert_body(x_ref, wg_ref, wu_ref, wd_ref, cmb_ref, out_ref):
    e = pl.program_id(0)
    x = x_ref[...]
    wg = wg_ref[0]
    wu = wu_ref[0]
    wd = wd_ref[0]
    g = jax.lax.dot_general(x, wg, (((1,), (1,)), ((), ())),
                            preferred_element_type=jnp.float32)
    u = jax.lax.dot_general(x, wu, (((1,), (1,)), ((), ())),
                            preferred_element_type=jnp.float32)
    act = g * jax.nn.sigmoid(g) * u
    y = jax.lax.dot_general(act, wd, (((1,), (1,)), ((), ())),
                            preferred_element_type=jnp.float32)
    cmb = cmb_ref[...]
    cols = jax.lax.broadcasted_iota(jnp.int32, cmb.shape, 1)
    w = jnp.sum(jnp.where(cols == e, cmb, 0.0), axis=1, keepdims=True)
    contrib = w * y

    @pl.when(e == 0)
    def _init():
        out_ref[...] = contrib

    @pl.when(e != 0)
    def _acc():
        out_ref[...] += contrib


def kernel(hidden_states, gate_w, Wg, Wu, Wd, Sg, Su, Sd):
    b, t, d = hidden_states.shape
    n = b * t
    e_r = gate_w.shape[0]
    i_dim = Wg.shape[1]
    flat = hidden_states.reshape(n, d)

    gwt = jnp.zeros((d, 128), jnp.float32).at[:, :e_r].set(gate_w.T)

    cmb, aux = pl.pallas_call(
        functools.partial(_router_body, n_experts=e_r, k_top=2),
        out_shape=(
            jax.ShapeDtypeStruct((n, 128), jnp.float32),
            jax.ShapeDtypeStruct((1, 1), jnp.float32),
        ),
        in_specs=[
            pl.BlockSpec((n, d), lambda: (0, 0)),
            pl.BlockSpec((d, 128), lambda: (0, 0)),
        ],
        out_specs=(
            pl.BlockSpec((n, 128), lambda: (0, 0)),
            pl.BlockSpec(memory_space=pltpu.SMEM),
        ),
    )(flat, gwt)

    n_tot = e_r + Sg.shape[0]
    wg_all = jnp.concatenate([Wg, Sg], axis=0)
    wu_all = jnp.concatenate([Wu, Su], axis=0)
    wd_all = jnp.concatenate([Wd, Sd], axis=0)
    cols = jax.lax.broadcasted_iota(jnp.int32, (n, 128), 1)
    cmb_all = jnp.where((cols >= e_r) & (cols < n_tot), 1.0, cmb)

    out = pl.pallas_call(
        _expert_body,
        grid=(n_tot,),
        out_shape=jax.ShapeDtypeStruct((n, d), jnp.float32),
        in_specs=[
            pl.BlockSpec((n, d), lambda e: (0, 0)),
            pl.BlockSpec((1, i_dim, d), lambda e: (e, 0, 0)),
            pl.BlockSpec((1, i_dim, d), lambda e: (e, 0, 0)),
            pl.BlockSpec((1, d, i_dim), lambda e: (e, 0, 0)),
            pl.BlockSpec((n, 128), lambda e: (0, 0)),
        ],
        out_specs=pl.BlockSpec((n, d), lambda e: (0, 0)),
    )(flat, wg_all, wu_all, wd_all, cmb_all)

    return out.reshape(b, t, d), aux[0, 0]
